# baseline (speedup 1.0000x reference)
import functools

import jax
import jax.numpy as jnp
from jax import lax
from jax.experimental import pallas as pl
from jax.experimental.pallas import tpu as pltpu
from jax.experimental.pallas import tpu_sc as plsc

NUM_FEATURES = 1000000
EMBED_DIM = 16
BATCH = 16384
FIELDS = 26

NC, NS, L = 2, 16, 16
NW = NC * NS
RPW = BATCH // NW
IDX_W = 128
ROWS_PW = RPW * FIELDS // IDX_W
CB = 64
G = CB * FIELDS
J = G // IDX_W
NCH = RPW // CB

VBLK = 128
NFULL = NUM_FEATURES // VBLK
OUT_W = 128
TILE_OUT = VBLK * EMBED_DIM // OUT_W
SLAB = 8
SLAB_V = SLAB * VBLK
SLAB_OUT = SLAB * TILE_OUT
NSLAB = NFULL // SLAB
PER_W = NSLAB // NW
EXTRA = NSLAB - PER_W * NW
NREST = NFULL - NSLAB * SLAB


def _tr_body(embt_hbm, tail_hbm, out_hbm, in_buf, stage, isem, osem):
    wid = lax.axis_index("s") * NC + lax.axis_index("c")
    n = PER_W + jnp.where(wid < EXTRA, 1, 0)
    start = wid * PER_W + jnp.minimum(wid, EXTRA)
    rows = lax.iota(jnp.int32, L)

    def in_copies(slot, sb, width=SLAB_V):
        return (
            pltpu.make_async_copy(
                embt_hbm.at[pl.ds(0, 8), pl.ds(sb * SLAB_V, width)],
                in_buf.at[slot, pl.ds(0, 8), pl.ds(0, width)], isem),
            pltpu.make_async_copy(
                embt_hbm.at[pl.ds(8, 8), pl.ds(sb * SLAB_V, width)],
                in_buf.at[slot, pl.ds(8, 8), pl.ds(0, width)], isem),
        )

    def out_copy(slot, sb, nrows=SLAB_OUT):
        return pltpu.make_async_copy(
            stage.at[slot, pl.ds(0, nrows)],
            out_hbm.at[pl.ds(sb * SLAB_OUT, nrows)], osem)

    def transpose(slot, nvo8):
        src = in_buf.at[slot]

        def tr_step(vo8, _):
            for k in range(8):
                vo = vo8 * 8 + k
                row = plsc.load_gather(
                    src, [rows, jnp.full((L,), vo, jnp.int32)])
                stage[slot, vo8, pl.ds(k * EMBED_DIM, EMBED_DIM)] = row
            return 0

        lax.fori_loop(0, nvo8, tr_step, 0)

    for cp in in_copies(0, start):
        cp.start()

    def step(i, _):
        slot = lax.rem(i, 2)
        sb = start + i
        for cp in in_copies(slot, sb):
            cp.wait()

        @pl.when(i + 1 < n)
        def _():
            for cp in in_copies(1 - slot, sb + 1):
                cp.start()

        @pl.when(i >= 2)
        def _():
            out_copy(slot, sb - 2).wait()

        transpose(slot, SLAB_V // 8)
        out_copy(slot, sb).start()
        return 0

    lax.fori_loop(0, n, step, 0)
    out_copy(0, start).wait()
    out_copy(1, start).wait()

    @pl.when(wid == 0)
    def _():
        for cp in in_copies(0, NSLAB, width=NREST * VBLK):
            cp.start()
        for cp in in_copies(0, NSLAB, width=NREST * VBLK):
            cp.wait()
        transpose(0, NREST * VBLK // 8)
        pltpu.sync_copy(stage.at[0, pl.ds(0, NREST * TILE_OUT)],
                        out_hbm.at[pl.ds(NSLAB * SLAB_OUT,
                                         NREST * TILE_OUT)])
        pltpu.sync_copy(tail_hbm, stage.at[0, pl.ds(0, 8)])
        pltpu.sync_copy(stage.at[0, pl.ds(0, 8)],
                        out_hbm.at[pl.ds(NFULL * TILE_OUT, 8)])


def _fm_body(idx_hbm, lin_hbm, emb_hbm, out_hbm,
             idx_all, emb_buf, lin_buf, out_buf, stage, gsem, lsem):
    wid = lax.axis_index("s") * NC + lax.axis_index("c")
    row0 = wid * ROWS_PW

    pltpu.sync_copy(idx_hbm.at[pl.ds(row0, ROWS_PW)], idx_all)

    def chunk_body(c, _):
        base = c * J
        copies = []
        for j in range(J):
            idx_row = idx_all.at[base + j]
            cp = pltpu.make_async_copy(
                emb_hbm.at[idx_row], emb_buf.at[pl.ds(j * IDX_W, IDX_W)],
                gsem)
            cp.start()
            copies.append(cp)
            cp = pltpu.make_async_copy(
                lin_hbm.at[idx_row], lin_buf.at[pl.ds(j * IDX_W, IDX_W)],
                lsem)
            cp.start()
            copies.append(cp)
        for cp in copies:
            cp.wait()

        def group_body(grp, _):
            r0 = grp * L

            def row_body(r, _):
                g0 = (r0 + r) * FIELDS
                s = jnp.zeros((L,), jnp.float32)
                q = jnp.zeros((L,), jnp.float32)
                for f in range(FIELDS):
                    e = emb_buf[g0 + f]
                    s = s + e
                    q = q + e * e
                lin_a = lin_buf[pl.ds(g0, L)]
                lin_b = lin_buf[pl.ds(g0 + L, L)]
                tail = jnp.where(lax.iota(jnp.int32, L) < (FIELDS - L),
                                 lin_b, jnp.zeros((L,), jnp.float32))
                stage[pl.ds(r * L, L)] = 0.5 * (s * s - q) + lin_a + tail
                return 0

            lax.fori_loop(0, L, row_body, 0)
            rows = lax.iota(jnp.int32, L) * L
            acc = jnp.zeros((L,), jnp.float32)
            for d in range(L):
                acc = acc + plsc.load_gather(stage, [rows + d])
            out_buf[pl.ds(c * CB + r0, L)] = acc
            return 0

        lax.fori_loop(0, CB // L, group_body, 0)
        return 0

    lax.fori_loop(0, NCH, chunk_body, 0)
    pltpu.sync_copy(out_buf, out_hbm.at[pl.ds(wid * RPW, RPW)])


def kernel(feature_indices, linear_w, embedding_w, bias):
    idx_flat = feature_indices.reshape(BATCH * FIELDS // IDX_W, IDX_W)
    lin = linear_w.reshape(NUM_FEATURES)
    embt = embedding_w.T
    tail = embedding_w[NFULL * VBLK:].reshape(8, OUT_W)

    mesh = plsc.VectorSubcoreMesh(core_axis_name="c", subcore_axis_name="s")

    tr = pl.kernel(
        _tr_body,
        out_type=jax.ShapeDtypeStruct(
            (NUM_FEATURES * EMBED_DIM // OUT_W, OUT_W), jnp.float32),
        mesh=mesh,
        compiler_params=pltpu.CompilerParams(
            needs_layout_passes=False, use_tc_tiling_on_sc=True),
        scratch_types=[
            pltpu.VMEM((2, L, SLAB_V), jnp.float32),
            pltpu.VMEM((2, SLAB_OUT, OUT_W), jnp.float32),
            pltpu.SemaphoreType.DMA,
            pltpu.SemaphoreType.DMA,
        ],
    )
    table = tr(embt, tail).reshape(NUM_FEATURES, EMBED_DIM)

    fm = pl.kernel(
        _fm_body,
        out_type=jax.ShapeDtypeStruct((BATCH,), jnp.float32),
        mesh=mesh,
        compiler_params=pltpu.CompilerParams(
            needs_layout_passes=False, use_tc_tiling_on_sc=False),
        scratch_types=[
            pltpu.VMEM((ROWS_PW, IDX_W), jnp.int32),
            pltpu.VMEM((G, EMBED_DIM), jnp.float32),
            pltpu.VMEM((G + L,), jnp.float32),
            pltpu.VMEM((RPW,), jnp.float32),
            pltpu.VMEM((L * L,), jnp.float32),
            pltpu.SemaphoreType.DMA,
            pltpu.SemaphoreType.DMA,
        ],
    )
    out = fm(idx_flat, lin, table)
    return out + bias


# speedup vs baseline: 2.1995x; 2.1995x over previous
import functools

import jax
import jax.numpy as jnp
from jax import lax
from jax.experimental import pallas as pl
from jax.experimental.pallas import tpu as pltpu
from jax.experimental.pallas import tpu_sc as plsc

NUM_FEATURES = 1000000
EMBED_DIM = 16
BATCH = 16384
FIELDS = 26

NC, NS, L = 2, 16, 16
NW = NC * NS
RPW = BATCH // NW
IDX_W = 128
ROWS_PW = RPW * FIELDS // IDX_W
CB = 64
G = CB * FIELDS
J = G // IDX_W
NCH = RPW // CB

VBLK = 128
NFULL = NUM_FEATURES // VBLK
OUT_W = 128
TILE_OUT = VBLK * EMBED_DIM // OUT_W
SLAB = 8
SLAB_V = SLAB * VBLK
SLAB_OUT = SLAB * TILE_OUT
NSLAB = NFULL // SLAB
PER_W = NSLAB // NW
EXTRA = NSLAB - PER_W * NW
NREST = NFULL - NSLAB * SLAB


def _tr_body(embt_hbm, tail_hbm, out_hbm, in_buf, stage, isem, osem):
    wid = lax.axis_index("s") * NC + lax.axis_index("c")
    n = PER_W + jnp.where(wid < EXTRA, 1, 0)
    start = wid * PER_W + jnp.minimum(wid, EXTRA)
    rows = lax.iota(jnp.int32, L)

    def in_copies(slot, sb, width=SLAB_V):
        return (
            pltpu.make_async_copy(
                embt_hbm.at[pl.ds(0, 8), pl.ds(sb * SLAB_V, width)],
                in_buf.at[slot, pl.ds(0, 8), pl.ds(0, width)], isem),
            pltpu.make_async_copy(
                embt_hbm.at[pl.ds(8, 8), pl.ds(sb * SLAB_V, width)],
                in_buf.at[slot, pl.ds(8, 8), pl.ds(0, width)], isem),
        )

    SLABW = SLAB_OUT * OUT_W

    def out_copy(slot, sb, nw=SLAB_OUT * OUT_W):
        return pltpu.make_async_copy(
            stage.at[pl.ds(slot * SLABW, nw)],
            out_hbm.at[pl.ds(sb * SLAB_OUT * OUT_W, nw)], osem)

    perms = [lax.rem(rows + k, L) for k in range(L)]

    def transpose(slot, nblk):
        src = in_buf.at[slot]
        base = slot * SLABW

        def tr_step(blk, _):
            v0 = blk * L
            for k in range(L):
                val = plsc.load_gather(src, [rows, perms[k] + v0])
                plsc.store_scatter(
                    stage, [base + (perms[k] + v0) * EMBED_DIM + rows], val)
            return 0

        lax.fori_loop(0, nblk, tr_step, 0)

    for cp in in_copies(0, start):
        cp.start()

    def step(i, _):
        slot = lax.rem(i, 2)
        sb = start + i
        for cp in in_copies(slot, sb):
            cp.wait()

        @pl.when(i + 1 < n)
        def _():
            for cp in in_copies(1 - slot, sb + 1):
                cp.start()

        @pl.when(i >= 2)
        def _():
            out_copy(slot, sb - 2).wait()

        transpose(slot, SLAB_V // L)
        out_copy(slot, sb).start()
        return 0

    lax.fori_loop(0, n, step, 0)
    out_copy(0, start).wait()
    out_copy(1, start).wait()

    @pl.when(wid == 0)
    def _():
        for cp in in_copies(0, NSLAB, width=NREST * VBLK):
            cp.start()
        for cp in in_copies(0, NSLAB, width=NREST * VBLK):
            cp.wait()
        transpose(0, NREST * VBLK // L)
        pltpu.sync_copy(
            stage.at[pl.ds(0, NREST * VBLK * EMBED_DIM)],
            out_hbm.at[pl.ds(NSLAB * SLAB_OUT * OUT_W,
                             NREST * VBLK * EMBED_DIM)])
        pltpu.sync_copy(tail_hbm, stage.at[pl.ds(0, 8 * OUT_W)])
        pltpu.sync_copy(stage.at[pl.ds(0, 8 * OUT_W)],
                        out_hbm.at[pl.ds(NFULL * TILE_OUT * OUT_W,
                                         8 * OUT_W)])


def _fm_body(idx_hbm, lin_hbm, emb_hbm, out_hbm,
             idx_all, emb_buf, lin_buf, out_buf, stage, gsem, lsem):
    wid = lax.axis_index("s") * NC + lax.axis_index("c")
    row0 = wid * ROWS_PW

    pltpu.sync_copy(idx_hbm.at[pl.ds(row0, ROWS_PW)], idx_all)

    def chunk_body(c, _):
        base = c * J
        copies = []
        for j in range(J):
            idx_row = idx_all.at[base + j]
            cp = pltpu.make_async_copy(
                emb_hbm.at[idx_row], emb_buf.at[pl.ds(j * IDX_W, IDX_W)],
                gsem)
            cp.start()
            copies.append(cp)
            cp = pltpu.make_async_copy(
                lin_hbm.at[idx_row], lin_buf.at[pl.ds(j * IDX_W, IDX_W)],
                lsem)
            cp.start()
            copies.append(cp)
        for cp in copies:
            cp.wait()

        def group_body(grp, _):
            r0 = grp * L

            def row_body(r, _):
                g0 = (r0 + r) * FIELDS
                s = jnp.zeros((L,), jnp.float32)
                q = jnp.zeros((L,), jnp.float32)
                for f in range(FIELDS):
                    e = emb_buf[g0 + f]
                    s = s + e
                    q = q + e * e
                lin_a = lin_buf[pl.ds(g0, L)]
                lin_b = lin_buf[pl.ds(g0 + L, L)]
                tail = jnp.where(lax.iota(jnp.int32, L) < (FIELDS - L),
                                 lin_b, jnp.zeros((L,), jnp.float32))
                stage[pl.ds(r * L, L)] = 0.5 * (s * s - q) + lin_a + tail
                return 0

            lax.fori_loop(0, L, row_body, 0)
            rows = lax.iota(jnp.int32, L) * L
            acc = jnp.zeros((L,), jnp.float32)
            for d in range(L):
                acc = acc + plsc.load_gather(stage, [rows + d])
            out_buf[pl.ds(c * CB + r0, L)] = acc
            return 0

        lax.fori_loop(0, CB // L, group_body, 0)
        return 0

    lax.fori_loop(0, NCH, chunk_body, 0)
    pltpu.sync_copy(out_buf, out_hbm.at[pl.ds(wid * RPW, RPW)])


def kernel(feature_indices, linear_w, embedding_w, bias):
    idx_flat = feature_indices.reshape(BATCH * FIELDS // IDX_W, IDX_W)
    lin = linear_w.reshape(NUM_FEATURES)
    embt = embedding_w.T
    tail = embedding_w[NFULL * VBLK:].reshape(8 * OUT_W)

    mesh = plsc.VectorSubcoreMesh(core_axis_name="c", subcore_axis_name="s")

    tr = pl.kernel(
        _tr_body,
        out_type=jax.ShapeDtypeStruct(
            (NUM_FEATURES * EMBED_DIM,), jnp.float32),
        mesh=mesh,
        compiler_params=pltpu.CompilerParams(
            needs_layout_passes=False, use_tc_tiling_on_sc=True),
        scratch_types=[
            pltpu.VMEM((2, L, SLAB_V), jnp.float32),
            pltpu.VMEM((2 * SLAB_OUT * OUT_W,), jnp.float32),
            pltpu.SemaphoreType.DMA,
            pltpu.SemaphoreType.DMA,
        ],
    )
    table = tr(embt, tail).reshape(NUM_FEATURES, EMBED_DIM)

    fm = pl.kernel(
        _fm_body,
        out_type=jax.ShapeDtypeStruct((BATCH,), jnp.float32),
        mesh=mesh,
        compiler_params=pltpu.CompilerParams(
            needs_layout_passes=False, use_tc_tiling_on_sc=False),
        scratch_types=[
            pltpu.VMEM((ROWS_PW, IDX_W), jnp.int32),
            pltpu.VMEM((G, EMBED_DIM), jnp.float32),
            pltpu.VMEM((G + L,), jnp.float32),
            pltpu.VMEM((RPW,), jnp.float32),
            pltpu.VMEM((L * L,), jnp.float32),
            pltpu.SemaphoreType.DMA,
            pltpu.SemaphoreType.DMA,
        ],
    )
    out = fm(idx_flat, lin, table)
    return out + bias


# speedup vs baseline: 2.3622x; 1.0739x over previous
import functools

import jax
import jax.numpy as jnp
from jax import lax
from jax.experimental import pallas as pl
from jax.experimental.pallas import tpu as pltpu
from jax.experimental.pallas import tpu_sc as plsc

NUM_FEATURES = 1000000
EMBED_DIM = 16
BATCH = 16384
FIELDS = 26

NC, NS, L = 2, 16, 16
NW = NC * NS
RPW = BATCH // NW
IDX_W = 128
ROWS_PW = RPW * FIELDS // IDX_W
CB = 64
G = CB * FIELDS
J = G // IDX_W
NCH = RPW // CB

VBLK = 128
NFULL = NUM_FEATURES // VBLK
OUT_W = 128
TILE_OUT = VBLK * EMBED_DIM // OUT_W
SLAB = 8
SLAB_V = SLAB * VBLK
SLAB_OUT = SLAB * TILE_OUT
NSLAB = NFULL // SLAB
PER_W = NSLAB // NW
EXTRA = NSLAB - PER_W * NW
NREST = NFULL - NSLAB * SLAB


def _tr_body(embt_hbm, tail_hbm, out_hbm, in_buf, stage, isem, osem):
    wid = lax.axis_index("s") * NC + lax.axis_index("c")
    n = PER_W + jnp.where(wid < EXTRA, 1, 0)
    start = wid * PER_W + jnp.minimum(wid, EXTRA)
    rows = lax.iota(jnp.int32, L)

    def in_copies(slot, sb, width=SLAB_V):
        return (
            pltpu.make_async_copy(
                embt_hbm.at[pl.ds(0, 8), pl.ds(sb * SLAB_V, width)],
                in_buf.at[slot, pl.ds(0, 8), pl.ds(0, width)], isem),
            pltpu.make_async_copy(
                embt_hbm.at[pl.ds(8, 8), pl.ds(sb * SLAB_V, width)],
                in_buf.at[slot, pl.ds(8, 8), pl.ds(0, width)], isem),
        )

    SLABW = SLAB_OUT * OUT_W

    def out_copy(slot, sb, nw=SLAB_OUT * OUT_W):
        return pltpu.make_async_copy(
            stage.at[pl.ds(slot * SLABW, nw)],
            out_hbm.at[pl.ds(sb * SLAB_OUT * OUT_W, nw)], osem)

    perms = [lax.rem(rows + k, L) for k in range(L)]

    def transpose(slot, nblk):
        src = in_buf.at[slot]
        base = slot * SLABW

        def tr_step(blk, _):
            v0 = blk * L
            for k in range(L):
                val = plsc.load_gather(src, [rows, perms[k] + v0])
                plsc.store_scatter(
                    stage, [base + (perms[k] + v0) * EMBED_DIM + rows], val)
            return 0

        lax.fori_loop(0, nblk, tr_step, 0)

    for cp in in_copies(0, start):
        cp.start()

    def step(i, _):
        slot = lax.rem(i, 2)
        sb = start + i
        for cp in in_copies(slot, sb):
            cp.wait()

        @pl.when(i + 1 < n)
        def _():
            for cp in in_copies(1 - slot, sb + 1):
                cp.start()

        @pl.when(i >= 2)
        def _():
            out_copy(slot, sb - 2).wait()

        transpose(slot, SLAB_V // L)
        out_copy(slot, sb).start()
        return 0

    lax.fori_loop(0, n, step, 0)
    out_copy(0, start).wait()
    out_copy(1, start).wait()

    @pl.when(wid == 0)
    def _():
        for cp in in_copies(0, NSLAB, width=NREST * VBLK):
            cp.start()
        for cp in in_copies(0, NSLAB, width=NREST * VBLK):
            cp.wait()
        transpose(0, NREST * VBLK // L)
        pltpu.sync_copy(
            stage.at[pl.ds(0, NREST * VBLK * EMBED_DIM)],
            out_hbm.at[pl.ds(NSLAB * SLAB_OUT * OUT_W,
                             NREST * VBLK * EMBED_DIM)])
        pltpu.sync_copy(tail_hbm, stage.at[pl.ds(0, 8 * OUT_W)])
        pltpu.sync_copy(stage.at[pl.ds(0, 8 * OUT_W)],
                        out_hbm.at[pl.ds(NFULL * TILE_OUT * OUT_W,
                                         8 * OUT_W)])


def _fm_body(idx_hbm, lin_hbm, emb_hbm, out_hbm,
             idx_all, emb_buf, lin_buf, out_buf, stage, gsem, lsem):
    wid = lax.axis_index("s") * NC + lax.axis_index("c")
    row0 = wid * ROWS_PW
    iota = lax.iota(jnp.int32, L)
    LB = G + L

    pltpu.sync_copy(idx_hbm.at[pl.ds(row0, ROWS_PW)], idx_all)

    def chunk_copies(c):
        slot = lax.rem(c, 2)
        base = c * J
        copies = []
        for j in range(J):
            idx_row = idx_all.at[base + j]
            copies.append(pltpu.make_async_copy(
                emb_hbm.at[idx_row],
                emb_buf.at[pl.ds(slot * G + j * IDX_W, IDX_W), :], gsem))
            copies.append(pltpu.make_async_copy(
                lin_hbm.at[idx_row],
                lin_buf.at[pl.ds(slot * LB + j * IDX_W, IDX_W)], lsem))
        return copies

    for cp in chunk_copies(0):
        cp.start()

    def chunk_body(c, _):
        slot = lax.rem(c, 2)
        for cp in chunk_copies(c):
            cp.wait()

        @pl.when(c + 1 < NCH)
        def _():
            for cp in chunk_copies(c + 1):
                cp.start()

        def group_body(grp, _):
            r0 = grp * L

            def row_body(r, _):
                g0 = slot * G + (r0 + r) * FIELDS
                l0 = slot * LB + (r0 + r) * FIELDS
                s = jnp.zeros((L,), jnp.float32)
                q = jnp.zeros((L,), jnp.float32)
                for f in range(FIELDS):
                    e = emb_buf[g0 + f]
                    s = s + e
                    q = q + e * e
                lin_a = lin_buf[pl.ds(l0, L)]
                lin_b = lin_buf[pl.ds(l0 + L, L)]
                tail = jnp.where(iota < (FIELDS - L), lin_b,
                                 jnp.zeros((L,), jnp.float32))
                stage[pl.ds(r * L, L)] = 0.5 * (s * s - q) + lin_a + tail
                return 0

            lax.fori_loop(0, L, row_body, 0)
            acc = jnp.zeros((L,), jnp.float32)
            for k in range(L):
                acc = acc + plsc.load_gather(
                    stage, [iota * L + lax.rem(iota + k, L)])
            out_buf[pl.ds(c * CB + r0, L)] = acc
            return 0

        lax.fori_loop(0, CB // L, group_body, 0)
        return 0

    lax.fori_loop(0, NCH, chunk_body, 0)
    pltpu.sync_copy(out_buf, out_hbm.at[pl.ds(wid * RPW, RPW)])


def kernel(feature_indices, linear_w, embedding_w, bias):
    idx_flat = feature_indices.reshape(BATCH * FIELDS // IDX_W, IDX_W)
    lin = linear_w.reshape(NUM_FEATURES)
    embt = embedding_w.T
    tail = embedding_w[NFULL * VBLK:].reshape(8 * OUT_W)

    mesh = plsc.VectorSubcoreMesh(core_axis_name="c", subcore_axis_name="s")

    tr = pl.kernel(
        _tr_body,
        out_type=jax.ShapeDtypeStruct(
            (NUM_FEATURES * EMBED_DIM,), jnp.float32),
        mesh=mesh,
        compiler_params=pltpu.CompilerParams(
            needs_layout_passes=False, use_tc_tiling_on_sc=True),
        scratch_types=[
            pltpu.VMEM((2, L, SLAB_V), jnp.float32),
            pltpu.VMEM((2 * SLAB_OUT * OUT_W,), jnp.float32),
            pltpu.SemaphoreType.DMA,
            pltpu.SemaphoreType.DMA,
        ],
    )
    table = tr(embt, tail).reshape(NUM_FEATURES, EMBED_DIM)

    fm = pl.kernel(
        _fm_body,
        out_type=jax.ShapeDtypeStruct((BATCH,), jnp.float32),
        mesh=mesh,
        compiler_params=pltpu.CompilerParams(
            needs_layout_passes=False, use_tc_tiling_on_sc=False),
        scratch_types=[
            pltpu.VMEM((ROWS_PW, IDX_W), jnp.int32),
            pltpu.VMEM((2 * G, EMBED_DIM), jnp.float32),
            pltpu.VMEM((2 * (G + L),), jnp.float32),
            pltpu.VMEM((RPW,), jnp.float32),
            pltpu.VMEM((L * L,), jnp.float32),
            pltpu.SemaphoreType.DMA,
            pltpu.SemaphoreType.DMA,
        ],
    )
    out = fm(idx_flat, lin, table)
    return out + bias
